# baseline (speedup 1.0000x reference)
import functools

import jax
import jax.numpy as jnp
from jax import lax
from jax.experimental import pallas as pl
from jax.experimental.pallas import tpu as pltpu
from jax.experimental.pallas import tpu_sc as plsc

NC = 2
NS = 16
NW = NC * NS
IW = 512
K = 1
GROUP = K * IW


def _emb_body(n_groups, D, token_hbm, table_hbm, out_hbm,
              idx_v, rows_v, gsem, ssem):
    wid = lax.axis_index("s") * NC + lax.axis_index("c")
    pltpu.sync_copy(token_hbm.at[wid], idx_v)

    def fire_gathers(h, b):
        for j in range(K):
            pltpu.async_copy(
                table_hbm.at[idx_v.at[h * K + j]],
                rows_v.at[b, pl.ds(j * IW, IW)],
                gsem.at[b])

    def drain_gathers(b):
        pltpu.make_async_copy(
            table_hbm.at[pl.ds(0, GROUP)], rows_v.at[b], gsem.at[b]).wait()

    def fire_store(h, b):
        pltpu.async_copy(rows_v.at[b], out_hbm.at[wid, h], ssem.at[b])

    def wait_store(b):
        pltpu.make_async_copy(
            rows_v.at[b], out_hbm.at[wid, 0], ssem.at[b]).wait()

    fire_gathers(0, 0)

    @pl.loop(0, n_groups, step=2)
    def _(g):
        @pl.when(g >= 1)
        def _():
            wait_store(1)
        fire_gathers(g + 1, 1)
        drain_gathers(0)
        fire_store(g, 0)

        wait_store(0)

        @pl.when(g + 2 < n_groups)
        def _():
            fire_gathers(g + 2, 0)
        drain_gathers(1)
        fire_store(g + 1, 1)

    wait_store(1)


def kernel(token_ids, E):
    B, S = token_ids.shape
    V, D = E.shape
    N = B * S

    b_per_w = N // NW
    n_groups = b_per_w // GROUP
    assert b_per_w * NW == N and n_groups * GROUP == b_per_w
    assert n_groups % 2 == 0

    flat = token_ids.reshape(NW, b_per_w // IW, IW).astype(jnp.int32)

    mesh = plsc.VectorSubcoreMesh(
        core_axis_name="c", subcore_axis_name="s", num_cores=NC,
        num_subcores=NS)

    run = functools.partial(
        pl.kernel,
        out_type=jax.ShapeDtypeStruct((NW, n_groups, GROUP, D), jnp.float32),
        mesh=mesh,
        compiler_params=pltpu.CompilerParams(use_tc_tiling_on_sc=False),
        scratch_types=[
            pltpu.VMEM((b_per_w // IW, IW), jnp.int32),
            pltpu.VMEM((2, GROUP, D), jnp.float32),
            pltpu.SemaphoreType.DMA((2,)),
            pltpu.SemaphoreType.DMA((2,)),
        ],
    )(functools.partial(_emb_body, n_groups, D))

    out = run(flat, E)
    return out.reshape(B, S, D)


# speedup vs baseline: 1.0023x; 1.0023x over previous
import functools

import jax
import jax.numpy as jnp
from jax import lax
from jax.experimental import pallas as pl
from jax.experimental.pallas import tpu as pltpu
from jax.experimental.pallas import tpu_sc as plsc

NC = 2
NS = 16
NW = NC * NS
BPG = 2


def _emb_body(batches_per_w, S, D, token_hbm, table_hbm, out_hbm,
              idx_v, rows_v, gsem, ssem):
    n_groups = batches_per_w // BPG
    wid = lax.axis_index("s") * NC + lax.axis_index("c")
    pltpu.sync_copy(token_hbm.at[wid], idx_v)

    def fire_gathers(h, b):
        for j in range(BPG):
            pltpu.async_copy(
                table_hbm.at[idx_v.at[h * BPG + j]],
                rows_v.at[b, j],
                gsem.at[b])

    def drain_gathers(b):
        for j in range(BPG):
            pltpu.make_async_copy(
                table_hbm.at[pl.ds(0, S)], rows_v.at[b, j], gsem.at[b]).wait()

    def fire_store(h, b):
        pltpu.async_copy(
            rows_v.at[b], out_hbm.at[pl.ds(wid * batches_per_w + h * BPG, BPG)],
            ssem.at[b])

    def wait_store(b):
        pltpu.make_async_copy(
            rows_v.at[b], out_hbm.at[pl.ds(0, BPG)], ssem.at[b]).wait()

    fire_gathers(0, 0)

    @pl.loop(0, n_groups, step=2)
    def _(g):
        @pl.when(g >= 1)
        def _():
            wait_store(1)
        fire_gathers(g + 1, 1)
        drain_gathers(0)
        fire_store(g, 0)

        wait_store(0)

        @pl.when(g + 2 < n_groups)
        def _():
            fire_gathers(g + 2, 0)
        drain_gathers(1)
        fire_store(g + 1, 1)

    wait_store(1)


def kernel(token_ids, E):
    B, S = token_ids.shape
    V, D = E.shape

    batches_per_w = B // NW
    assert batches_per_w * NW == B and batches_per_w % BPG == 0
    assert (batches_per_w // BPG) % 2 == 0

    tok = token_ids.reshape(NW, batches_per_w, S).astype(jnp.int32)

    mesh = plsc.VectorSubcoreMesh(
        core_axis_name="c", subcore_axis_name="s", num_cores=NC,
        num_subcores=NS)

    run = functools.partial(
        pl.kernel,
        out_type=jax.ShapeDtypeStruct((B, S, D), jnp.float32),
        mesh=mesh,
        compiler_params=pltpu.CompilerParams(use_tc_tiling_on_sc=False),
        scratch_types=[
            pltpu.VMEM((batches_per_w, S), jnp.int32),
            pltpu.VMEM((2, BPG, S, D), jnp.float32),
            pltpu.SemaphoreType.DMA((2,)),
            pltpu.SemaphoreType.DMA((2,)),
        ],
    )(functools.partial(_emb_body, batches_per_w, S, D))

    return run(tok, E)
